# baseline (speedup 1.0000x reference)
import jax
import jax.numpy as jnp
from jax.experimental import pallas as pl
from jax.experimental.pallas import tpu as pltpu

BATCH = 128
NCAT = 100000
RB = 8
NRB = BATCH // RB

CW = 1024
NPAIR = NCAT // (2 * CW)
NFULL = 2 * NPAIR
TAIL = NCAT - NFULL * CW

_K2 = 1234
_KS2 = _K2 ^ 0x1BD11BDA
_ROT0 = (13, 15, 26, 6)
_ROT1 = (17, 29, 16, 24)


def _rotl(x, d):
    return (x << jnp.uint32(d)) | (x >> jnp.uint32(32 - d))


def _threefry_bits(x1):
    x0 = x1
    x1 = _rotl(x1, 13) ^ x0
    for r in _ROT0[1:]:
        x0 = x0 + x1
        x1 = _rotl(x1, r) ^ x0
    x0 = x0 + jnp.uint32(_K2)
    x1 = x1 + jnp.uint32(_KS2 + 1)
    for r in _ROT1:
        x0 = x0 + x1
        x1 = _rotl(x1, r) ^ x0
    x0 = x0 + jnp.uint32(_KS2)
    x1 = x1 + jnp.uint32(2)
    for r in _ROT0:
        x0 = x0 + x1
        x1 = _rotl(x1, r) ^ x0
    x1 = x1 + jnp.uint32(_K2 + 3)
    for r in _ROT1:
        x0 = x0 + x1
        x1 = _rotl(x1, r) ^ x0
    x0 = x0 + jnp.uint32(_K2)
    x1 = x1 + jnp.uint32(_KS2 + 4)
    for r in _ROT0:
        x0 = x0 + x1
        x1 = _rotl(x1, r) ^ x0
    x0 = x0 + jnp.uint32(_KS2)
    x1 = x1 + jnp.uint32(5)
    return x0 ^ x1


def _gumbel_from_bits(bits):
    fb = (bits >> jnp.uint32(9)) | jnp.uint32(0x3F800000)
    floats = jax.lax.bitcast_convert_type(fb, jnp.float32) - jnp.float32(1.0)
    u = jnp.maximum(jnp.float32(1.1754943508222875e-38), floats)
    return -jnp.log(-jnp.log(u))


def _body(x_ref, out_ref, idx_scr, za_scr, ca_scr, zb_scr, cb_scr):
    s = pl.program_id(0)

    @pl.when(s < NRB)
    def _argmax():
        row = s * RB + jax.lax.broadcasted_iota(jnp.int32, (RB, CW), 0)
        basep = row * NCAT + jnp.int32(_K2)
        col0 = jax.lax.broadcasted_iota(jnp.int32, (RB, CW), 1)

        za_scr[...] = jnp.full((RB, CW), -jnp.inf, jnp.float32)
        ca_scr[...] = jnp.zeros((RB, CW), jnp.int32)
        zb_scr[...] = jnp.full((RB, CW), -jnp.inf, jnp.float32)
        cb_scr[...] = jnp.zeros((RB, CW), jnp.int32)

        def half(j, z_scr, c_scr):
            col = j * CW + col0
            x1 = (basep + col).astype(jnp.uint32)
            x = x_ref[:, pl.ds(j * CW, CW)]
            z = x + _gumbel_from_bits(_threefry_bits(x1))
            rz = z_scr[...]
            better = z > rz
            z_scr[...] = jnp.where(better, z, rz)
            c_scr[...] = jnp.where(better, col, c_scr[...])

        def body(p, _):
            half(2 * p, za_scr, ca_scr)
            half(2 * p + 1, zb_scr, cb_scr)
            return 0

        jax.lax.fori_loop(0, NPAIR, body, 0)

        run_za, run_ca = za_scr[...], ca_scr[...]
        run_zb, run_cb = zb_scr[...], cb_scr[...]

        colt = NFULL * CW + jax.lax.broadcasted_iota(jnp.int32, (RB, TAIL), 1)
        rowt = s * RB + jax.lax.broadcasted_iota(jnp.int32, (RB, TAIL), 0)
        x1t = (rowt * NCAT + jnp.int32(_K2) + colt).astype(jnp.uint32)
        xt = x_ref[:, pl.ds(NFULL * CW, TAIL)]
        zt = xt + _gumbel_from_bits(_threefry_bits(x1t))

        big = jnp.int32(2**31 - 1)
        rmax = jnp.maximum(
            jnp.maximum(jnp.max(run_za, axis=1, keepdims=True),
                        jnp.max(run_zb, axis=1, keepdims=True)),
            jnp.max(zt, axis=1, keepdims=True))
        cand_a = jnp.min(jnp.where(run_za == rmax, run_ca, big),
                         axis=1, keepdims=True)
        cand_b = jnp.min(jnp.where(run_zb == rmax, run_cb, big),
                         axis=1, keepdims=True)
        cand_t = jnp.min(jnp.where(zt == rmax, colt, big),
                         axis=1, keepdims=True)
        idx_scr[pl.ds(s * RB, RB), :] = jnp.minimum(
            jnp.minimum(cand_a, cand_b), cand_t)

    @pl.when(s > 0)
    def _onehot():
        idx = idx_scr[pl.ds((s - 1) * RB, RB), :]
        col = jax.lax.broadcasted_iota(jnp.int32, (RB, NCAT), 1)
        out_ref[...] = (col == idx).astype(jnp.float32)


@jax.jit
def kernel(dist_params):
    out = pl.pallas_call(
        _body,
        grid=(NRB + 1,),
        in_specs=[pl.BlockSpec((RB, NCAT), lambda s: (jnp.minimum(s, NRB - 1), 0))],
        out_specs=pl.BlockSpec((RB, NCAT), lambda s: (jnp.maximum(s - 1, 0), 0)),
        out_shape=jax.ShapeDtypeStruct((BATCH, NCAT), jnp.float32),
        scratch_shapes=[
            pltpu.VMEM((BATCH, 1), jnp.int32),
            pltpu.VMEM((RB, CW), jnp.float32),
            pltpu.VMEM((RB, CW), jnp.int32),
            pltpu.VMEM((RB, CW), jnp.float32),
            pltpu.VMEM((RB, CW), jnp.int32),
        ],
    )(dist_params)
    return out
